# baseline (speedup 1.0000x reference)
import jax
import jax.numpy as jnp
from jax import lax
from jax.experimental import pallas as pl
from jax.experimental.pallas import tpu as pltpu
from jax.experimental.pallas import tpu_sc as plsc

_EMB_DIM = 128
_BATCH = 4096
_PAD = 100352
_NSUB = 16
_EPT = 128
_SLICE = _PAD // _NSUB
_GROUPS = _EMB_DIM // 16


def _sc_body(posI, negI, ent, rel,
             main_o, esum_o, ecnt_o, rsum_o, rcnt_o,
             idx_p, idx_n,
             hp, rp, tp, hn, rn, tn,
             vh_p, vt_p, vr_p, vh_n, vt_n,
             ones_v, mbuf, zbuf,
             esum_s, ecnt_s, rsum_s, rcnt_s,
             sem_g, sem_i, sem_s):
    s = lax.axis_index("s")

    zero16 = jnp.zeros((16,), jnp.float32)
    one16 = jnp.ones((16,), jnp.float32)
    lane = lax.iota(jnp.int32, 16)
    last = lane == 15

    pltpu.sync_copy(posI.at[s], idx_p)
    pltpu.sync_copy(negI.at[s], idx_n)
    gathers = [
        pltpu.async_copy(ent.at[idx_p.at[0]], hp, sem_g),
        pltpu.async_copy(rel.at[idx_p.at[1]], rp, sem_g),
        pltpu.async_copy(ent.at[idx_p.at[2]], tp, sem_g),
        pltpu.async_copy(ent.at[idx_n.at[0]], hn, sem_g),
        pltpu.async_copy(rel.at[idx_n.at[1]], rn, sem_g),
        pltpu.async_copy(ent.at[idx_n.at[2]], tn, sem_g),
    ]

    def zfill(i, carry):
        zbuf[pl.ds(i * 16, 16)] = zero16
        return carry

    lax.fori_loop(0, _SLICE // 64, zfill, None)
    off = s * _SLICE
    inits = []
    for arr in (esum_s, ecnt_s, rsum_s, rcnt_s):
        for q in range(4):
            inits.append(pltpu.async_copy(
                zbuf, arr.at[pl.ds(off + q * (_SLICE // 4), _SLICE // 4)],
                sem_i))
    for i in range(_EPT // 16):
        ones_v[pl.ds(i * 16, 16)] = one16

    gdn = lax.GatherDimensionNumbers(
        offset_dims=(), collapsed_slice_dims=(0,), start_index_map=(0,))

    def hsum(x):
        for k in (1, 2, 4, 8):
            perm = lax.gather(x, (lane ^ k)[:, None], gdn, slice_sizes=(1,),
                              mode=lax.GatherScatterMode.PROMISE_IN_BOUNDS)
            x = x + perm
        return x

    def vsqrt(x):
        i = plsc.bitcast(x, jnp.int32)
        y = plsc.bitcast(jnp.int32(0x1FBD1DF5) + (i >> 1), jnp.float32)
        for _ in range(3):
            y = 0.5 * (y + x / y)
        return y

    for g in gathers:
        g.wait()
    for i in inits:
        i.wait()

    def edge_body(e, macc):
        dps = dns = hps = tps = rps = hns = tns = zero16
        for j in range(_GROUPS):
            col = pl.ds(j * 16, 16)
            hpv = hp[e, col]
            rpv = rp[e, col]
            tpv = tp[e, col]
            hnv = hn[e, col]
            rnv = rn[e, col]
            tnv = tn[e, col]
            dp = hpv + rpv - tpv
            dn = hnv + rnv - tnv
            dps = dps + dp * dp
            dns = dns + dn * dn
            hps = hps + hpv * hpv
            tps = tps + tpv * tpv
            rps = rps + rpv * rpv
            hns = hns + hnv * hnv
            tns = tns + tnv * tnv
        eidx = jnp.full((16,), e, jnp.int32)
        plsc.store_scatter(vh_p, [eidx], hsum(hps), mask=last)
        plsc.store_scatter(vt_p, [eidx], hsum(tps), mask=last)
        plsc.store_scatter(vr_p, [eidx], hsum(rps), mask=last)
        plsc.store_scatter(vh_n, [eidx], hsum(hns), mask=last)
        plsc.store_scatter(vt_n, [eidx], hsum(tns), mask=last)
        contrib = jnp.maximum(1.0 + vsqrt(hsum(dps)) - vsqrt(hsum(dns)), 0.0)
        return macc + jnp.where(last, contrib, 0.0)

    macc = lax.fori_loop(0, _EPT, edge_body, zero16)
    mbuf[pl.ds(0, 16)] = macc
    m0 = pltpu.async_copy(mbuf, main_o.at[s], sem_i)

    plsc.subcore_barrier()

    scatters = [
        pltpu.async_copy(vh_p, esum_s.at[idx_p.at[0]], sem_s, add=True),
        pltpu.async_copy(ones_v, ecnt_s.at[idx_p.at[0]], sem_s, add=True),
        pltpu.async_copy(vt_p, esum_s.at[idx_p.at[2]], sem_s, add=True),
        pltpu.async_copy(ones_v, ecnt_s.at[idx_p.at[2]], sem_s, add=True),
        pltpu.async_copy(vh_n, esum_s.at[idx_n.at[0]], sem_s, add=True),
        pltpu.async_copy(ones_v, ecnt_s.at[idx_n.at[0]], sem_s, add=True),
        pltpu.async_copy(vt_n, esum_s.at[idx_n.at[2]], sem_s, add=True),
        pltpu.async_copy(ones_v, ecnt_s.at[idx_n.at[2]], sem_s, add=True),
        pltpu.async_copy(vr_p, rsum_s.at[idx_p.at[1]], sem_s, add=True),
        pltpu.async_copy(ones_v, rcnt_s.at[idx_p.at[1]], sem_s, add=True),
    ]
    for sd in scatters:
        sd.wait()
    m0.wait()

    plsc.subcore_barrier()

    pltpu.sync_copy(esum_s.at[pl.ds(off, _SLICE)], esum_o.at[pl.ds(off, _SLICE)])
    pltpu.sync_copy(ecnt_s.at[pl.ds(off, _SLICE)], ecnt_o.at[pl.ds(off, _SLICE)])
    pltpu.sync_copy(rsum_s.at[pl.ds(off, _SLICE)], rsum_o.at[pl.ds(off, _SLICE)])
    pltpu.sync_copy(rcnt_s.at[pl.ds(off, _SLICE)], rcnt_o.at[pl.ds(off, _SLICE)])


def _tc_reduce(moa, mob, esa, esb, eca, ecb, rsa, rsb, rca, rcb, out):
    main = jnp.sum(moa[...]) + jnp.sum(mob[...])

    def scale_loss(sa, sb, ca, cb):
        tot = sa[...] + sb[...]
        cnt = ca[...] + cb[...]
        pres = cnt > 0.5
        val = jnp.sqrt(tot / jnp.maximum(cnt, 1.0)) - 1.0
        num = jnp.sum(jnp.where(pres, jnp.maximum(val, 0.0), 0.0))
        den = jnp.sum(jnp.where(pres, 1.0, 0.0))
        return num / den

    total = (main + scale_loss(esa, esb, eca, ecb)
             + scale_loss(rsa, rsb, rca, rcb))
    out[...] = jnp.reshape(total, (1, 1))


def _make_half():
    mesh = plsc.VectorSubcoreMesh(
        core_axis_name="c", subcore_axis_name="s", num_cores=1)
    f32 = jnp.float32
    return pl.kernel(
        _sc_body,
        out_type=[
            jax.ShapeDtypeStruct((_NSUB, 16), f32),
            jax.ShapeDtypeStruct((_PAD,), f32),
            jax.ShapeDtypeStruct((_PAD,), f32),
            jax.ShapeDtypeStruct((_PAD,), f32),
            jax.ShapeDtypeStruct((_PAD,), f32),
        ],
        mesh=mesh,
        compiler_params=pltpu.CompilerParams(needs_layout_passes=False),
        scratch_types=[
            pltpu.VMEM((3, _EPT), jnp.int32),
            pltpu.VMEM((3, _EPT), jnp.int32),
            pltpu.VMEM((_EPT, _EMB_DIM), f32),
            pltpu.VMEM((_EPT, _EMB_DIM), f32),
            pltpu.VMEM((_EPT, _EMB_DIM), f32),
            pltpu.VMEM((_EPT, _EMB_DIM), f32),
            pltpu.VMEM((_EPT, _EMB_DIM), f32),
            pltpu.VMEM((_EPT, _EMB_DIM), f32),
            pltpu.VMEM((_EPT,), f32),
            pltpu.VMEM((_EPT,), f32),
            pltpu.VMEM((_EPT,), f32),
            pltpu.VMEM((_EPT,), f32),
            pltpu.VMEM((_EPT,), f32),
            pltpu.VMEM((_EPT,), f32),
            pltpu.VMEM((16,), f32),
            pltpu.VMEM((_SLICE // 4,), f32),
            pltpu.VMEM_SHARED((_PAD,), f32),
            pltpu.VMEM_SHARED((_PAD,), f32),
            pltpu.VMEM_SHARED((_PAD,), f32),
            pltpu.VMEM_SHARED((_PAD,), f32),
            pltpu.SemaphoreType.DMA,
            pltpu.SemaphoreType.DMA,
            pltpu.SemaphoreType.DMA,
        ],
    )


@jax.jit
def _impl(pos_edge, neg_edge, entity_emb, relation_emb):
    nblk = 2 * _NSUB
    posI = jnp.asarray(pos_edge, jnp.int32).T.reshape(3, nblk, _EPT)
    posI = posI.transpose(1, 0, 2)
    negI = jnp.asarray(neg_edge, jnp.int32).T.reshape(3, nblk, _EPT)
    negI = negI.transpose(1, 0, 2)

    half = _make_half()
    moa, esa, eca, rsa, rca = half(
        posI[:_NSUB], negI[:_NSUB], entity_emb, relation_emb)
    mob, esb, ecb, rsb, rcb = half(
        posI[_NSUB:], negI[_NSUB:], entity_emb, relation_emb)

    f32 = jnp.float32
    red = pl.pallas_call(
        _tc_reduce,
        out_shape=jax.ShapeDtypeStruct((1, 1), f32),
    )
    h = _PAD // 128
    loss = red(
        moa, mob,
        esa.reshape(h, 128), esb.reshape(h, 128),
        eca.reshape(h, 128), ecb.reshape(h, 128),
        rsa.reshape(h, 128), rsb.reshape(h, 128),
        rca.reshape(h, 128), rcb.reshape(h, 128),
    )
    return jnp.reshape(loss, ())


def kernel(pos_edge, neg_edge, entity_emb, relation_emb):
    return _impl(pos_edge, neg_edge, entity_emb, relation_emb)


# speedup vs baseline: 1.4653x; 1.4653x over previous
import jax
import jax.numpy as jnp
from jax import lax
from jax.experimental import pallas as pl
from jax.experimental.pallas import tpu as pltpu
from jax.experimental.pallas import tpu_sc as plsc

_EMB_DIM = 128
_BATCH = 4096
_PAD = 100352
_TILES = 32
_EPT = _BATCH // _TILES
_NBLK = _EPT // 16
_SLICE = _PAD // 16
_GROUPS = _EMB_DIM // 16


def _sc_body(posI, negI, ent, rel,
             main_o, esum_o, ecnt_o, rsum_o, rcnt_o,
             idx_p, idx_n,
             hp, rp, tp, hn, rn, tn,
             vh_p, vt_p, vr_p, vh_n, vt_n, v_dp,
             p0, p1, p2, p3,
             ones_v, mbuf, zbuf,
             esum_s, ecnt_s, rsum_s, rcnt_s,
             sem_g, sem_i, sem_s):
    c = lax.axis_index("c")
    s = lax.axis_index("s")
    wid = c * 16 + s

    zero16 = jnp.zeros((16,), jnp.float32)
    one16 = jnp.ones((16,), jnp.float32)
    lane = lax.iota(jnp.int32, 16)
    lane17 = lane * 17

    pltpu.sync_copy(posI.at[wid], idx_p)
    pltpu.sync_copy(negI.at[wid], idx_n)
    g0 = pltpu.async_copy(ent.at[idx_p.at[0]], hp, sem_g)
    g1 = pltpu.async_copy(rel.at[idx_p.at[1]], rp, sem_g)
    g2 = pltpu.async_copy(ent.at[idx_p.at[2]], tp, sem_g)
    g3 = pltpu.async_copy(ent.at[idx_n.at[0]], hn, sem_g)
    g4 = pltpu.async_copy(rel.at[idx_n.at[1]], rn, sem_g)
    g5 = pltpu.async_copy(ent.at[idx_n.at[2]], tn, sem_g)

    def zfill(i, carry):
        zbuf[pl.ds(i * 16, 16)] = zero16
        return carry

    lax.fori_loop(0, _SLICE // 64, zfill, None)
    off = s * _SLICE
    inits = []
    for arr in (esum_s, ecnt_s, rsum_s, rcnt_s):
        for q in range(4):
            inits.append(pltpu.async_copy(
                zbuf, arr.at[pl.ds(off + q * (_SLICE // 4), _SLICE // 4)],
                sem_i))
    for i in range(_EPT // 16):
        ones_v[pl.ds(i * 16, 16)] = one16

    def tsum(pbuf):
        acc = plsc.load_gather(pbuf, [lane17])
        for cc in range(1, 16):
            acc = acc + plsc.load_gather(pbuf, [lane17 + cc])
        return acc

    def vsqrt(x):
        i = plsc.bitcast(x, jnp.int32)
        y = plsc.bitcast(jnp.int32(0x1FBD1DF5) + (i >> 1), jnp.float32)
        for _ in range(3):
            y = 0.5 * (y + x / y)
        return y

    for g in (g0, g1, g2):
        g.wait()

    def pos_block(blk, carry):
        def pos_edge(el, carry2):
            e = blk * 16 + el
            dacc = hacc = tacc = racc = zero16
            for j in range(_GROUPS):
                col = pl.ds(j * 16, 16)
                hv = hp[e, col]
                rv = rp[e, col]
                tv = tp[e, col]
                d = hv + rv - tv
                dacc = dacc + d * d
                hacc = hacc + hv * hv
                tacc = tacc + tv * tv
                racc = racc + rv * rv
            base = el * 17
            p0[pl.ds(base, 16)] = dacc
            p1[pl.ds(base, 16)] = hacc
            p2[pl.ds(base, 16)] = tacc
            p3[pl.ds(base, 16)] = racc
            return carry2

        lax.fori_loop(0, 16, pos_edge, None)
        row = pl.ds(blk * 16, 16)
        v_dp[row] = vsqrt(tsum(p0))
        vh_p[row] = tsum(p1)
        vt_p[row] = tsum(p2)
        vr_p[row] = tsum(p3)
        return carry

    lax.fori_loop(0, _NBLK, pos_block, None)

    for i in inits:
        i.wait()
    plsc.subcore_barrier()

    pos_scatters = [
        pltpu.async_copy(vh_p, esum_s.at[idx_p.at[0]], sem_s, add=True),
        pltpu.async_copy(ones_v, ecnt_s.at[idx_p.at[0]], sem_s, add=True),
        pltpu.async_copy(vt_p, esum_s.at[idx_p.at[2]], sem_s, add=True),
        pltpu.async_copy(ones_v, ecnt_s.at[idx_p.at[2]], sem_s, add=True),
        pltpu.async_copy(vr_p, rsum_s.at[idx_p.at[1]], sem_s, add=True),
        pltpu.async_copy(ones_v, rcnt_s.at[idx_p.at[1]], sem_s, add=True),
    ]

    for g in (g3, g4, g5):
        g.wait()

    def neg_block(blk, macc):
        def neg_edge(el, carry2):
            e = blk * 16 + el
            dacc = hacc = tacc = zero16
            for j in range(_GROUPS):
                col = pl.ds(j * 16, 16)
                hv = hn[e, col]
                rv = rn[e, col]
                tv = tn[e, col]
                d = hv + rv - tv
                dacc = dacc + d * d
                hacc = hacc + hv * hv
                tacc = tacc + tv * tv
            base = el * 17
            p0[pl.ds(base, 16)] = dacc
            p1[pl.ds(base, 16)] = hacc
            p2[pl.ds(base, 16)] = tacc
            return carry2

        lax.fori_loop(0, 16, neg_edge, None)
        row = pl.ds(blk * 16, 16)
        sn = vsqrt(tsum(p0))
        vh_n[row] = tsum(p1)
        vt_n[row] = tsum(p2)
        contrib = jnp.maximum(1.0 + v_dp[row] - sn, 0.0)
        return macc + contrib

    macc = lax.fori_loop(0, _NBLK, neg_block, zero16)
    mbuf[pl.ds(0, 16)] = macc
    m0 = pltpu.async_copy(mbuf, main_o.at[wid], sem_i)

    neg_scatters = [
        pltpu.async_copy(vh_n, esum_s.at[idx_n.at[0]], sem_s, add=True),
        pltpu.async_copy(ones_v, ecnt_s.at[idx_n.at[0]], sem_s, add=True),
        pltpu.async_copy(vt_n, esum_s.at[idx_n.at[2]], sem_s, add=True),
        pltpu.async_copy(ones_v, ecnt_s.at[idx_n.at[2]], sem_s, add=True),
    ]
    for sd in pos_scatters + neg_scatters:
        sd.wait()
    m0.wait()

    plsc.subcore_barrier()

    pltpu.sync_copy(esum_s.at[pl.ds(off, _SLICE)], esum_o.at[c, pl.ds(off, _SLICE)])
    pltpu.sync_copy(ecnt_s.at[pl.ds(off, _SLICE)], ecnt_o.at[c, pl.ds(off, _SLICE)])
    pltpu.sync_copy(rsum_s.at[pl.ds(off, _SLICE)], rsum_o.at[c, pl.ds(off, _SLICE)])
    pltpu.sync_copy(rcnt_s.at[pl.ds(off, _SLICE)], rcnt_o.at[c, pl.ds(off, _SLICE)])


def _tc_reduce(mo, es, ec, rs, rc, out):
    main = jnp.sum(mo[...])

    def scale_loss(sum_ref, cnt_ref):
        tot = sum_ref[0] + sum_ref[1]
        cnt = cnt_ref[0] + cnt_ref[1]
        pres = cnt > 0.5
        val = jnp.sqrt(tot / jnp.maximum(cnt, 1.0)) - 1.0
        num = jnp.sum(jnp.where(pres, jnp.maximum(val, 0.0), 0.0))
        den = jnp.sum(jnp.where(pres, 1.0, 0.0))
        return num / den

    total = main + scale_loss(es, ec) + scale_loss(rs, rc)
    out[...] = jnp.reshape(total, (1, 1))


@jax.jit
def _impl(pos_edge, neg_edge, entity_emb, relation_emb):
    posI = jnp.asarray(pos_edge, jnp.int32).T.reshape(3, _TILES, _EPT)
    posI = posI.transpose(1, 0, 2)
    negI = jnp.asarray(neg_edge, jnp.int32).T.reshape(3, _TILES, _EPT)
    negI = negI.transpose(1, 0, 2)

    mesh = plsc.VectorSubcoreMesh(core_axis_name="c", subcore_axis_name="s")
    f32 = jnp.float32
    sc = pl.kernel(
        _sc_body,
        out_type=[
            jax.ShapeDtypeStruct((_TILES, 16), f32),
            jax.ShapeDtypeStruct((2, _PAD), f32),
            jax.ShapeDtypeStruct((2, _PAD), f32),
            jax.ShapeDtypeStruct((2, _PAD), f32),
            jax.ShapeDtypeStruct((2, _PAD), f32),
        ],
        mesh=mesh,
        compiler_params=pltpu.CompilerParams(needs_layout_passes=False),
        scratch_types=[
            pltpu.VMEM((3, _EPT), jnp.int32),
            pltpu.VMEM((3, _EPT), jnp.int32),
            pltpu.VMEM((_EPT, _EMB_DIM), f32),
            pltpu.VMEM((_EPT, _EMB_DIM), f32),
            pltpu.VMEM((_EPT, _EMB_DIM), f32),
            pltpu.VMEM((_EPT, _EMB_DIM), f32),
            pltpu.VMEM((_EPT, _EMB_DIM), f32),
            pltpu.VMEM((_EPT, _EMB_DIM), f32),
            pltpu.VMEM((_EPT,), f32),
            pltpu.VMEM((_EPT,), f32),
            pltpu.VMEM((_EPT,), f32),
            pltpu.VMEM((_EPT,), f32),
            pltpu.VMEM((_EPT,), f32),
            pltpu.VMEM((_EPT,), f32),
            pltpu.VMEM((17 * 16,), f32),
            pltpu.VMEM((17 * 16,), f32),
            pltpu.VMEM((17 * 16,), f32),
            pltpu.VMEM((17 * 16,), f32),
            pltpu.VMEM((_EPT,), f32),
            pltpu.VMEM((16,), f32),
            pltpu.VMEM((_SLICE // 4,), f32),
            pltpu.VMEM_SHARED((_PAD,), f32),
            pltpu.VMEM_SHARED((_PAD,), f32),
            pltpu.VMEM_SHARED((_PAD,), f32),
            pltpu.VMEM_SHARED((_PAD,), f32),
            pltpu.SemaphoreType.DMA,
            pltpu.SemaphoreType.DMA,
            pltpu.SemaphoreType.DMA,
        ],
    )
    mo, es, ec, rs, rc = sc(posI, negI, entity_emb, relation_emb)

    red = pl.pallas_call(
        _tc_reduce,
        out_shape=jax.ShapeDtypeStruct((1, 1), f32),
    )
    loss = red(
        mo,
        es.reshape(2, _PAD // 128, 128), ec.reshape(2, _PAD // 128, 128),
        rs.reshape(2, _PAD // 128, 128), rc.reshape(2, _PAD // 128, 128),
    )
    return jnp.reshape(loss, ())


def kernel(pos_edge, neg_edge, entity_emb, relation_emb):
    return _impl(pos_edge, neg_edge, entity_emb, relation_emb)
